# baseline (speedup 1.0000x reference)
import functools

import jax
import jax.numpy as jnp
from jax import lax
from jax.experimental import pallas as pl
from jax.experimental.pallas import tpu as pltpu
from jax.experimental.pallas import tpu_sc as plsc

NEG_SLOPE = 0.01
_N = 10000
_NP = 10240
_E = 320000
_D = 128
_NC = 2
_NS = 16
_NW = _NC * _NS
_CPW = _E // _NW
_K = 100
_ROWS = _CPW // _K
_VOFF = (0, 16, 32, 48, 64, 80, 84)
_NPS = _NP // _NS
_CK = 5
_NCH = _ROWS // _CK


def _prep_body(fs_ref, fd_ref, al_ref, ar_ref, el_ref, er_ref, m16_ref):
    el = jnp.sum(fs_ref[...] * al_ref[...], axis=-1)
    er = jnp.sum(fd_ref[...] * ar_ref[...], axis=-1)
    el_ref[...] = el
    er_ref[...] = er
    m16_ref[...] = jnp.full((16,), jnp.max(el), jnp.float32)


def _finish_body(accp_ref, denp_ref, out_ref):
    dsum = jnp.sum(denp_ref[...], axis=(0, 1))
    r = 1.0 / jnp.maximum(dsum[:_N], 1e-16)
    out_ref[...] = (accp_ref[0] + accp_ref[1]) * r[:, None]


def _sc_body(elr_hbm, m16_hbm, dst_hbm, srcnp_hbm, feat_hbm,
             denp_hbm, accp_hbm,
             dst_c, srcnp_c,
             rows_a, rows_b, elg_a, elg_b, erg_a, erg_b,
             exb_a, exb_b, m16_v, zrow_v,
             elr_sh, den_sh, acc_sh,
             semg_a, semg_b, seml_a, seml_b, semw_a, semw_b, semd_a, semd_b,
             semi):
    cid = lax.axis_index("c")
    sid = lax.axis_index("s")
    wid = cid * _NS + sid

    buf_a = (rows_a, elg_a, erg_a, exb_a, semg_a, seml_a, semw_a, semd_a)
    buf_b = (rows_b, elg_b, erg_b, exb_b, semg_b, seml_b, semw_b, semd_b)

    def dst_row(b):
        return dst_c.at[(b // _CK) % 2, b % _CK]

    def srcnp_row(b):
        return srcnp_c.at[(b // _CK) % 2, b % _CK]

    pltpu.sync_copy(dst_hbm.at[wid, 0], dst_c.at[0])
    pltpu.sync_copy(srcnp_hbm.at[wid, 0], srcnp_c.at[0])
    pltpu.sync_copy(m16_hbm, m16_v)
    mv = m16_v[...]

    nsl = pl.ds(sid * _NPS, _NPS)
    nsl2 = pl.ds(_NP + sid * _NPS, _NPS)
    pltpu.sync_copy(elr_hbm.at[nsl], elr_sh.at[nsl])
    pltpu.sync_copy(elr_hbm.at[nsl2], elr_sh.at[nsl2])

    zeros16 = jnp.zeros((16,), jnp.float32)

    @pl.loop(0, (_K * _D) // 16)
    def _(t):
        rows_a[t // (_D // 16), pl.ds((t % (_D // 16)) * 16, 16)] = zeros16

    @pl.loop(0, _NPS // 16)
    def _(t):
        zrow_v[pl.ds(t * 16, 16)] = zeros16

    pltpu.sync_copy(zrow_v, den_sh.at[nsl])

    @pl.when(sid < _NS - 1)
    def _():
        @pl.loop(0, 6)
        def _(q):
            pltpu.sync_copy(rows_a, acc_sh.at[pl.ds(sid * 640 + q * _K, _K), :])

        pltpu.sync_copy(rows_a.at[pl.ds(0, 40), :],
                        acc_sh.at[pl.ds(sid * 640 + 600, 40), :])

    @pl.when(sid == _NS - 1)
    def _():
        @pl.loop(0, 4)
        def _(q):
            pltpu.sync_copy(rows_a, acc_sh.at[pl.ds(9600 + q * _K, _K), :])

    plsc.subcore_barrier()

    def issue_batch(b, buf):
        rows_v, elg_v, erg_v, _, semg, seml, _, _ = buf

        @pl.when(b < _ROWS)
        def _():
            pltpu.async_copy(feat_hbm.at[srcnp_row(b)], rows_v, semg)
            pltpu.async_copy(elr_sh.at[srcnp_row(b)], elg_v, seml)
            pltpu.async_copy(elr_sh.at[dst_row(b)], erg_v, seml)

    def phase(b, cur, nxt):
        rows_v, elg_v, erg_v, exb_v, semg, seml, semw, semd = cur

        @pl.when(b < _ROWS)
        def _():
            pltpu.make_async_copy(
                elr_sh.at[srcnp_row(b)], elg_v, seml).wait()
            pltpu.make_async_copy(
                elr_sh.at[dst_row(b)], erg_v, seml).wait()
            @pl.when(b >= 2)
            def _():
                pltpu.make_async_copy(
                    exb_v, den_sh.at[dst_row(0)], semd).wait()

            for off in _VOFF:
                sl = pl.ds(off, 16)
                erg = erg_v[sl]
                e = elg_v[sl] + erg
                e = jnp.where(e > 0, e, NEG_SLOPE * e)
                t = mv + erg
                mp = jnp.where(t > 0, t, NEG_SLOPE * t)
                exb_v[sl] = jnp.exp(e - mp)
            pltpu.async_copy(exb_v, den_sh.at[dst_row(b)], semd, add=True)

        @pl.when(jnp.logical_and(b % _CK == _CK - 5, b + 5 < _ROWS))
        def _():
            c = b // _CK + 1
            pltpu.async_copy(dst_hbm.at[wid, c], dst_c.at[c % 2], semi)
            pltpu.async_copy(srcnp_hbm.at[wid, c], srcnp_c.at[c % 2], semi)

        @pl.when(jnp.logical_and((b + 1) % _CK == 0, b + 1 < _ROWS))
        def _():
            c = (b + 1) // _CK
            pltpu.make_async_copy(
                dst_hbm.at[wid, c], dst_c.at[c % 2], semi).wait()
            pltpu.make_async_copy(
                srcnp_hbm.at[wid, c], srcnp_c.at[c % 2], semi).wait()

        @pl.when(jnp.logical_and(b >= 1, b <= _ROWS))
        def _():
            pltpu.make_async_copy(
                nxt[0], acc_sh.at[dst_row(0)], nxt[6]).wait()

        issue_batch(b + 1, nxt)

        @pl.when(b < _ROWS)
        def _():
            pltpu.make_async_copy(
                feat_hbm.at[srcnp_row(b)], rows_v, semg).wait()

            @pl.loop(0, _K, unroll=8)
            def _(i):
                idx16 = jnp.broadcast_to(i, (16,)).astype(jnp.int32)
                a = plsc.load_gather(exb_v, [idx16])
                for j in range(_D // 16):
                    rows_v[i, pl.ds(j * 16, 16)] = (
                        rows_v[i, pl.ds(j * 16, 16)] * a)

            pltpu.async_copy(rows_v, acc_sh.at[dst_row(b)], semw, add=True)

    issue_batch(0, buf_a)

    @pl.loop(0, _ROWS + 1, step=2)
    def _(g):
        phase(g, buf_a, buf_b)
        phase(g + 1, buf_b, buf_a)

    pltpu.make_async_copy(exb_a, den_sh.at[dst_row(0)], semd_a).wait()
    pltpu.make_async_copy(exb_b, den_sh.at[dst_row(0)], semd_b).wait()

    plsc.subcore_barrier()

    @pl.when(sid < _NS - 1)
    def _():
        osl = pl.ds(sid * 640, 640)
        pltpu.sync_copy(acc_sh.at[osl, :], accp_hbm.at[cid, osl, :])

    @pl.when(sid == _NS - 1)
    def _():
        osl = pl.ds(9600, 400)
        pltpu.sync_copy(acc_sh.at[osl, :], accp_hbm.at[cid, osl, :])

    @pl.when(sid == 0)
    def _():
        pltpu.sync_copy(den_sh, denp_hbm.at[cid, 0])


@functools.cache
def _make_sc_call():
    return pl.kernel(
        _sc_body,
        out_type=(
            jax.ShapeDtypeStruct((_NC, 1, _NP), jnp.float32),
            jax.ShapeDtypeStruct((_NC, _N, _D), jnp.float32),
        ),
        mesh=plsc.VectorSubcoreMesh(core_axis_name="c", subcore_axis_name="s"),
        compiler_params=pltpu.CompilerParams(needs_layout_passes=False),
        scratch_types=[
            pltpu.VMEM((2, _CK, _K), jnp.int32),
            pltpu.VMEM((2, _CK, _K), jnp.int32),
            pltpu.VMEM((_K, _D), jnp.float32),
            pltpu.VMEM((_K, _D), jnp.float32),
            pltpu.VMEM((_K,), jnp.float32),
            pltpu.VMEM((_K,), jnp.float32),
            pltpu.VMEM((_K,), jnp.float32),
            pltpu.VMEM((_K,), jnp.float32),
            pltpu.VMEM((_K,), jnp.float32),
            pltpu.VMEM((_K,), jnp.float32),
            pltpu.VMEM((16,), jnp.float32),
            pltpu.VMEM((_NPS,), jnp.float32),
            pltpu.VMEM_SHARED((2 * _NP,), jnp.float32),
            pltpu.VMEM_SHARED((_NP,), jnp.float32),
            pltpu.VMEM_SHARED((_N, _D), jnp.float32),
            pltpu.SemaphoreType.DMA,
            pltpu.SemaphoreType.DMA,
            pltpu.SemaphoreType.DMA,
            pltpu.SemaphoreType.DMA,
            pltpu.SemaphoreType.DMA,
            pltpu.SemaphoreType.DMA,
            pltpu.SemaphoreType.DMA,
            pltpu.SemaphoreType.DMA,
            pltpu.SemaphoreType.DMA,
        ],
    )


def kernel(feat_src, feat_dst, edge_index, attn_l, attn_r):
    N, D = feat_src.shape
    el, er, m16 = pl.pallas_call(
        _prep_body,
        out_shape=(
            jax.ShapeDtypeStruct((N,), jnp.float32),
            jax.ShapeDtypeStruct((N,), jnp.float32),
            jax.ShapeDtypeStruct((16,), jnp.float32),
        ),
    )(feat_src, feat_dst, attn_l, attn_r)
    pad = (0, _NP - _N)
    elr = jnp.concatenate([jnp.pad(er, pad), jnp.pad(el, pad)])
    src4 = edge_index[0].reshape(_NW, _NCH, _CK, _K)
    dst4 = edge_index[1].reshape(_NW, _NCH, _CK, _K)
    srcnp4 = src4 + _NP
    feat2 = jnp.pad(feat_src, ((_NP, 0), (0, 0)))
    denp, accp = _make_sc_call()(elr, m16, dst4, srcnp4, feat2)
    out = pl.pallas_call(
        _finish_body,
        out_shape=jax.ShapeDtypeStruct((N, D), jnp.float32),
    )(accp, denp)
    return out


# speedup vs baseline: 1.0462x; 1.0462x over previous
import functools

import jax
import jax.numpy as jnp
from jax import lax
from jax.experimental import pallas as pl
from jax.experimental.pallas import tpu as pltpu
from jax.experimental.pallas import tpu_sc as plsc

NEG_SLOPE = 0.01
_N = 10000
_NP = 10240
_E = 320000
_D = 128
_NC = 2
_NS = 16
_NW = _NC * _NS
_CPW = _E // _NW
_K = 80
_ROWS = _CPW // _K
_VPB = _K // 16
_NPS = _NP // _NS
_CK = 25
_NCH = _ROWS // _CK


def _prep_body(fs_ref, fd_ref, al_ref, ar_ref, el_ref, er_ref, m16_ref):
    el = jnp.sum(fs_ref[...] * al_ref[...], axis=-1)
    er = jnp.sum(fd_ref[...] * ar_ref[...], axis=-1)
    el_ref[...] = el
    er_ref[...] = er
    m16_ref[...] = jnp.full((16,), jnp.max(el), jnp.float32)


def _finish_body(accp_ref, denp_ref, out_ref):
    dsum = jnp.sum(denp_ref[...], axis=(0, 1))
    r = 1.0 / jnp.maximum(dsum[:_N], 1e-16)
    out_ref[...] = (accp_ref[0] + accp_ref[1]) * r[:, None]


def _sc_body(el_hbm, er_hbm, m16_hbm, src_hbm, dst_hbm, feat_hbm,
             denp_hbm, accp_hbm,
             src_c, dst_c,
             rows_a, rows_b, elg_a, elg_b, erg_a, erg_b,
             exb_a, exb_b, m16_v, zrow_v,
             el_sh, er_sh, den_sh, acc_sh,
             semg_a, semg_b, seml_a, seml_b, semw_a, semw_b, semd_a, semd_b,
             semi):
    cid = lax.axis_index("c")
    sid = lax.axis_index("s")
    wid = cid * _NS + sid

    buf_a = (rows_a, elg_a, erg_a, exb_a, semg_a, seml_a, semw_a, semd_a)
    buf_b = (rows_b, elg_b, erg_b, exb_b, semg_b, seml_b, semw_b, semd_b)

    def src_row(b):
        return src_c.at[(b // _CK) % 2, b % _CK]

    def dst_row(b):
        return dst_c.at[(b // _CK) % 2, b % _CK]

    pltpu.sync_copy(src_hbm.at[wid, 0], src_c.at[0])
    pltpu.sync_copy(dst_hbm.at[wid, 0], dst_c.at[0])
    pltpu.sync_copy(m16_hbm, m16_v)
    mv = m16_v[...]

    nsl = pl.ds(sid * _NPS, _NPS)
    pltpu.sync_copy(el_hbm.at[nsl], el_sh.at[nsl])
    pltpu.sync_copy(er_hbm.at[nsl], er_sh.at[nsl])

    zeros16 = jnp.zeros((16,), jnp.float32)

    @pl.loop(0, (_K * _D) // 16)
    def _(t):
        rows_a[t // (_D // 16), pl.ds((t % (_D // 16)) * 16, 16)] = zeros16

    @pl.loop(0, _NPS // 16)
    def _(t):
        zrow_v[pl.ds(t * 16, 16)] = zeros16

    pltpu.sync_copy(zrow_v, den_sh.at[nsl])

    @pl.when(sid < _NS - 1)
    def _():
        @pl.loop(0, 640 // _K)
        def _(q):
            pltpu.sync_copy(rows_a, acc_sh.at[pl.ds(sid * 640 + q * _K, _K), :])

    @pl.when(sid == _NS - 1)
    def _():
        @pl.loop(0, 400 // _K)
        def _(q):
            pltpu.sync_copy(rows_a, acc_sh.at[pl.ds(9600 + q * _K, _K), :])

    plsc.subcore_barrier()

    def issue_batch(b, buf):
        rows_v, elg_v, erg_v, _, semg, seml, _, _ = buf

        @pl.when(b < _ROWS)
        def _():
            pltpu.async_copy(feat_hbm.at[src_row(b)], rows_v, semg)
            pltpu.async_copy(el_sh.at[src_row(b)], elg_v, seml)
            pltpu.async_copy(er_sh.at[dst_row(b)], erg_v, seml)

    def phase(b, cur, nxt):
        rows_v, elg_v, erg_v, exb_v, semg, seml, semw, semd = cur

        @pl.when(b < _ROWS)
        def _():
            pltpu.make_async_copy(el_sh.at[src_row(b)], elg_v, seml).wait()
            pltpu.make_async_copy(er_sh.at[dst_row(b)], erg_v, seml).wait()
            @pl.when(b >= 2)
            def _():
                pltpu.make_async_copy(
                    exb_v, den_sh.at[dst_row(0)], semd).wait()

            for j in range(_VPB):
                sl = pl.ds(j * 16, 16)
                erg = erg_v[sl]
                e = elg_v[sl] + erg
                e = jnp.where(e > 0, e, NEG_SLOPE * e)
                t = mv + erg
                mp = jnp.where(t > 0, t, NEG_SLOPE * t)
                exb_v[sl] = jnp.exp(e - mp)
            pltpu.async_copy(exb_v, den_sh.at[dst_row(b)], semd, add=True)

        @pl.when(jnp.logical_and(b % _CK == _CK - 5, b + 5 < _ROWS))
        def _():
            c = b // _CK + 1
            pltpu.async_copy(src_hbm.at[wid, c], src_c.at[c % 2], semi)
            pltpu.async_copy(dst_hbm.at[wid, c], dst_c.at[c % 2], semi)

        @pl.when(jnp.logical_and((b + 1) % _CK == 0, b + 1 < _ROWS))
        def _():
            c = (b + 1) // _CK
            pltpu.make_async_copy(
                src_hbm.at[wid, c], src_c.at[c % 2], semi).wait()
            pltpu.make_async_copy(
                dst_hbm.at[wid, c], dst_c.at[c % 2], semi).wait()

        @pl.when(jnp.logical_and(b >= 1, b <= _ROWS))
        def _():
            pltpu.make_async_copy(
                nxt[0], acc_sh.at[dst_row(0)], nxt[6]).wait()

        issue_batch(b + 1, nxt)

        @pl.when(b < _ROWS)
        def _():
            pltpu.make_async_copy(
                feat_hbm.at[src_row(b)], rows_v, semg).wait()

            @pl.loop(0, _K, unroll=8)
            def _(i):
                idx16 = jnp.broadcast_to(i, (16,)).astype(jnp.int32)
                a = plsc.load_gather(exb_v, [idx16])
                for j in range(_D // 16):
                    rows_v[i, pl.ds(j * 16, 16)] = (
                        rows_v[i, pl.ds(j * 16, 16)] * a)

            pltpu.async_copy(rows_v, acc_sh.at[dst_row(b)], semw, add=True)

    issue_batch(0, buf_a)

    @pl.loop(0, _ROWS + 1, step=2)
    def _(g):
        phase(g, buf_a, buf_b)
        phase(g + 1, buf_b, buf_a)

    pltpu.make_async_copy(exb_a, den_sh.at[dst_row(0)], semd_a).wait()
    pltpu.make_async_copy(exb_b, den_sh.at[dst_row(0)], semd_b).wait()

    plsc.subcore_barrier()

    @pl.when(sid < _NS - 1)
    def _():
        osl = pl.ds(sid * 640, 640)
        pltpu.sync_copy(acc_sh.at[osl, :], accp_hbm.at[cid, osl, :])

    @pl.when(sid == _NS - 1)
    def _():
        osl = pl.ds(9600, 400)
        pltpu.sync_copy(acc_sh.at[osl, :], accp_hbm.at[cid, osl, :])

    @pl.when(sid == 0)
    def _():
        pltpu.sync_copy(den_sh, denp_hbm.at[cid, 0])


@functools.cache
def _make_sc_call():
    return pl.kernel(
        _sc_body,
        out_type=(
            jax.ShapeDtypeStruct((_NC, 1, _NP), jnp.float32),
            jax.ShapeDtypeStruct((_NC, _N, _D), jnp.float32),
        ),
        mesh=plsc.VectorSubcoreMesh(core_axis_name="c", subcore_axis_name="s"),
        compiler_params=pltpu.CompilerParams(needs_layout_passes=False),
        scratch_types=[
            pltpu.VMEM((2, _CK, _K), jnp.int32),
            pltpu.VMEM((2, _CK, _K), jnp.int32),
            pltpu.VMEM((_K, _D), jnp.float32),
            pltpu.VMEM((_K, _D), jnp.float32),
            pltpu.VMEM((_K,), jnp.float32),
            pltpu.VMEM((_K,), jnp.float32),
            pltpu.VMEM((_K,), jnp.float32),
            pltpu.VMEM((_K,), jnp.float32),
            pltpu.VMEM((_K,), jnp.float32),
            pltpu.VMEM((_K,), jnp.float32),
            pltpu.VMEM((16,), jnp.float32),
            pltpu.VMEM((_NPS,), jnp.float32),
            pltpu.VMEM_SHARED((_NP,), jnp.float32),
            pltpu.VMEM_SHARED((_NP,), jnp.float32),
            pltpu.VMEM_SHARED((_NP,), jnp.float32),
            pltpu.VMEM_SHARED((_N, _D), jnp.float32),
            pltpu.SemaphoreType.DMA,
            pltpu.SemaphoreType.DMA,
            pltpu.SemaphoreType.DMA,
            pltpu.SemaphoreType.DMA,
            pltpu.SemaphoreType.DMA,
            pltpu.SemaphoreType.DMA,
            pltpu.SemaphoreType.DMA,
            pltpu.SemaphoreType.DMA,
            pltpu.SemaphoreType.DMA,
        ],
    )


def kernel(feat_src, feat_dst, edge_index, attn_l, attn_r):
    N, D = feat_src.shape
    el, er, m16 = pl.pallas_call(
        _prep_body,
        out_shape=(
            jax.ShapeDtypeStruct((N,), jnp.float32),
            jax.ShapeDtypeStruct((N,), jnp.float32),
            jax.ShapeDtypeStruct((16,), jnp.float32),
        ),
    )(feat_src, feat_dst, attn_l, attn_r)
    pad = (0, _NP - _N)
    el = jnp.pad(el, pad)
    er = jnp.pad(er, pad)
    src4 = edge_index[0].reshape(_NW, _NCH, _CK, _K)
    dst4 = edge_index[1].reshape(_NW, _NCH, _CK, _K)
    denp, accp = _make_sc_call()(el, er, m16, src4, dst4, feat_src)
    out = pl.pallas_call(
        _finish_body,
        out_shape=jax.ShapeDtypeStruct((N, D), jnp.float32),
    )(accp, denp)
    return out
